# baseline (speedup 1.0000x reference)
import functools

import jax
import jax.numpy as jnp
from jax import lax
from jax.experimental import pallas as pl
from jax.experimental.pallas import tpu as pltpu
from jax.experimental.pallas import tpu_sc as plsc

T = 2048
D = 1024
H = 1024
O = 1024
E = 8
K = 2

BLK = 256
NB = (T * K) // BLK + E
NP = NB * BLK

SC_NC = 2
SC_NS = 16
NW = SC_NC * SC_NS

_SC_MESH = plsc.VectorSubcoreMesh(
    core_axis_name="c", subcore_axis_name="s",
    num_cores=SC_NC, num_subcores=SC_NS)

GROWS = NP // NW
GCHUNK = 48
GNCH = GROWS // GCHUNK
CROWS = T // NW
CCHUNK = 16
CNCH = CROWS // CCHUNK


def _dispatch_body(x_hbm, tok_hbm, xs_hbm, idx_v, buf0, buf1, gs0, gs1, ws0,
                   ws1):
    wid = lax.axis_index("s") * SC_NC + lax.axis_index("c")
    base = wid * GROWS
    bufs, gsems, wsems = (buf0, buf1), (gs0, gs1), (ws0, ws1)
    for c in range(GNCH):
        pltpu.sync_copy(tok_hbm.at[pl.ds(base + c * GCHUNK, GCHUNK)],
                        idx_v.at[c])
    gd, wd = [None] * GNCH, [None] * GNCH
    gd[0] = pltpu.async_copy(x_hbm.at[idx_v.at[0]], bufs[0], gsems[0])
    for c in range(1, GNCH + 1):
        if c < GNCH:
            if c >= 2:
                wd[c - 2].wait()
            gd[c] = pltpu.async_copy(
                x_hbm.at[idx_v.at[c]], bufs[c % 2], gsems[c % 2])
        p = c - 1
        gd[p].wait()
        wd[p] = pltpu.async_copy(
            bufs[p % 2], xs_hbm.at[pl.ds(base + p * GCHUNK, GCHUNK), :],
            wsems[p % 2])
    wd[GNCH - 2].wait()
    wd[GNCH - 1].wait()


def _sc_dispatch(x, row_token):
    return pl.kernel(
        _dispatch_body,
        out_type=jax.ShapeDtypeStruct((NP, D), jnp.float32),
        mesh=_SC_MESH,
        scratch_types=[
            pltpu.VMEM((GNCH, GCHUNK), jnp.int32),
            pltpu.VMEM((GCHUNK, D), jnp.float32),
            pltpu.VMEM((GCHUNK, D), jnp.float32),
            pltpu.SemaphoreType.DMA,
            pltpu.SemaphoreType.DMA,
            pltpu.SemaphoreType.DMA,
            pltpu.SemaphoreType.DMA,
        ],
    )(x, row_token)


def _combine_body(eo_hbm, pa_hbm, pb_hbm, out_hbm, ia_v, ib_v, a0, a1, b0, b1,
                  o0, o1, gs0, gs1, ws0, ws1):
    wid = lax.axis_index("s") * SC_NC + lax.axis_index("c")
    base = wid * CROWS
    abufs, bbufs, obufs = (a0, a1), (b0, b1), (o0, o1)
    gsems, wsems = (gs0, gs1), (ws0, ws1)
    pltpu.sync_copy(pa_hbm.at[pl.ds(base, CROWS)], ia_v)
    pltpu.sync_copy(pb_hbm.at[pl.ds(base, CROWS)], ib_v)
    ga, gb, wo = [None] * CNCH, [None] * CNCH, [None] * CNCH

    def start_gathers(c):
        ga[c] = pltpu.async_copy(
            eo_hbm.at[ia_v.at[pl.ds(c * CCHUNK, CCHUNK)]],
            abufs[c % 2], gsems[c % 2])
        gb[c] = pltpu.async_copy(
            eo_hbm.at[ib_v.at[pl.ds(c * CCHUNK, CCHUNK)]],
            bbufs[c % 2], gsems[c % 2])

    start_gathers(0)
    for c in range(1, CNCH + 1):
        if c < CNCH:
            if c >= 2:
                wo[c - 2].wait()
            start_gathers(c)
        p = c - 1
        ga[p].wait()
        gb[p].wait()
        av, bv, ov = abufs[p % 2], bbufs[p % 2], obufs[p % 2]

        @plsc.parallel_loop(0, CCHUNK * (O // 16), 1, unroll=8)
        def _add(i):
            r = lax.shift_right_logical(i, 6)
            col = pl.multiple_of(
                lax.shift_left(jnp.bitwise_and(i, 63), 4), 16)
            sl = pl.ds(col, 16)
            ov[r, sl] = av[r, sl] + bv[r, sl]

        wo[p] = pltpu.async_copy(
            ov, out_hbm.at[pl.ds(base + p * CCHUNK, CCHUNK), :],
            wsems[p % 2])
    wo[CNCH - 2].wait()
    wo[CNCH - 1].wait()


def _sc_combine(eo_sorted, pos_a, pos_b):
    return pl.kernel(
        _combine_body,
        out_type=jax.ShapeDtypeStruct((T, O), jnp.float32),
        mesh=_SC_MESH,
        scratch_types=[
            pltpu.VMEM((CROWS,), jnp.int32),
            pltpu.VMEM((CROWS,), jnp.int32),
            pltpu.VMEM((CCHUNK, O), jnp.float32),
            pltpu.VMEM((CCHUNK, O), jnp.float32),
            pltpu.VMEM((CCHUNK, O), jnp.float32),
            pltpu.VMEM((CCHUNK, O), jnp.float32),
            pltpu.VMEM((CCHUNK, O), jnp.float32),
            pltpu.VMEM((CCHUNK, O), jnp.float32),
            pltpu.SemaphoreType.DMA,
            pltpu.SemaphoreType.DMA,
            pltpu.SemaphoreType.DMA,
            pltpu.SemaphoreType.DMA,
        ],
    )(eo_sorted, pos_a, pos_b)


def _gmm_body(be_ref, x_ref, g_ref, w1_ref, b1_ref, w2_ref, b2_ref, w3_ref,
              b3_ref, o_ref):
    @pl.when(pl.program_id(0) < be_ref[NB])
    def _():
        x = x_ref[...]
        h1 = jnp.dot(x, w1_ref[0], preferred_element_type=jnp.float32)
        h1 = jnp.maximum(h1 + b1_ref[0], 0.0)
        h2 = jnp.dot(h1, w2_ref[0], preferred_element_type=jnp.float32)
        h2 = jnp.maximum(h2 + b2_ref[0], 0.0)
        eo = jnp.dot(h2, w3_ref[0], preferred_element_type=jnp.float32)
        eo = eo + b3_ref[0]
        o_ref[...] = eo * g_ref[:, 0:1]


def _gmm(block_expert, x_sorted, gates_mat, W1, b1, W2, b2, W3, b3):
    def rows_map(i, be):
        return (i, 0)

    def w_map(i, be):
        return (be[i], 0, 0)

    def b_map(i, be):
        return (be[i], 0, 0)

    grid_spec = pltpu.PrefetchScalarGridSpec(
        num_scalar_prefetch=1,
        grid=(NB,),
        in_specs=[
            pl.BlockSpec((BLK, D), rows_map),
            pl.BlockSpec((BLK, 128), rows_map),
            pl.BlockSpec((1, D, H), w_map),
            pl.BlockSpec((1, 1, H), b_map),
            pl.BlockSpec((1, H, H), w_map),
            pl.BlockSpec((1, 1, H), b_map),
            pl.BlockSpec((1, H, O), w_map),
            pl.BlockSpec((1, 1, O), b_map),
        ],
        out_specs=pl.BlockSpec((BLK, O), rows_map),
    )
    return pl.pallas_call(
        _gmm_body,
        grid_spec=grid_spec,
        out_shape=jax.ShapeDtypeStruct((NP, O), jnp.float32),
    )(block_expert, x_sorted, gates_mat,
      W1, b1.reshape(E, 1, H), W2, b2.reshape(E, 1, H), W3, b3.reshape(E, 1, O))


BLKR = 512


def _router_body(x_ref, wr_ref, br_ref, ls_ref, out_ref, cnt_ref, carry):
    i = pl.program_id(0)

    @pl.when(i == 0)
    def _():
        carry[...] = jnp.zeros_like(carry)

    logits = jnp.dot(x_ref[...], wr_ref[...],
                     preferred_element_type=jnp.float32) + br_ref[...]
    li = lax.broadcasted_iota(jnp.int32, (BLKR, 128), 1)
    m1 = jnp.max(logits, axis=1)
    a1 = jnp.min(jnp.where(logits == m1[:, None], li, 128), axis=1)
    l2 = jnp.where(li == a1[:, None], -jnp.inf, logits)
    m2 = jnp.max(l2, axis=1)
    a2 = jnp.min(jnp.where(l2 == m2[:, None], li, 128), axis=1)
    z = jnp.sum(jnp.exp(logits - m1[:, None]), axis=1)
    p1 = 1.0 / z
    p2 = jnp.exp(m2 - m1) / z
    den = p1 + p2 + 1e-6
    wn1 = p1 / den
    wn2 = p2 / den
    oh0 = (li == a1[:, None]).astype(jnp.float32)
    oh1 = (li == a2[:, None]).astype(jnp.float32)
    s = jnp.dot(ls_ref[...], oh0 + oh1,
                preferred_element_type=jnp.float32) + carry[0:1, :]
    r0 = jnp.sum(s * oh0, axis=1)
    r1 = jnp.sum((s + oh0) * oh1, axis=1)
    colsum = jnp.sum(oh0 + oh1, axis=0, keepdims=True)
    carry[...] = carry[...] + colsum
    out = jnp.where(li == 0, wn1[:, None],
          jnp.where(li == 1, wn2[:, None],
          jnp.where(li == 2, a1[:, None].astype(jnp.float32),
          jnp.where(li == 3, a2[:, None].astype(jnp.float32),
          jnp.where(li == 4, r0[:, None],
          jnp.where(li == 5, r1[:, None], 0.0))))))
    out_ref[...] = out

    @pl.when(i == pl.num_programs(0) - 1)
    def _():
        cnt_ref[...] = carry[...]


def _router(x, Wr, br):
    wr_pad = jnp.zeros((D, 128), jnp.float32).at[:, :E].set(Wr)
    br_pad = jnp.full((1, 128), -1e30, jnp.float32).at[0, :E].set(br)
    ls = jnp.tril(jnp.ones((BLKR, BLKR), jnp.float32), -1)
    grid_spec = pltpu.PrefetchScalarGridSpec(
        num_scalar_prefetch=0,
        grid=(T // BLKR,),
        in_specs=[
            pl.BlockSpec((BLKR, D), lambda i: (i, 0)),
            pl.BlockSpec((D, 128), lambda i: (0, 0)),
            pl.BlockSpec((1, 128), lambda i: (0, 0)),
            pl.BlockSpec((BLKR, BLKR), lambda i: (0, 0)),
        ],
        out_specs=[
            pl.BlockSpec((BLKR, 128), lambda i: (i, 0)),
            pl.BlockSpec((1, 128), lambda i: (0, 0)),
        ],
        scratch_shapes=[pltpu.VMEM((1, 128), jnp.float32)],
    )
    return pl.pallas_call(
        _router_body,
        grid_spec=grid_spec,
        out_shape=[
            jax.ShapeDtypeStruct((T, 128), jnp.float32),
            jax.ShapeDtypeStruct((1, 128), jnp.float32),
        ],
    )(x, wr_pad, br_pad, ls)


def kernel(x, Wr, br, W1, b1, W2, b2, W3, b3):
    rout, cnt = _router(x, Wr, br)
    wn1, wn2 = rout[:, 0], rout[:, 1]
    e0 = rout[:, 2].astype(jnp.int32)
    e1 = rout[:, 3].astype(jnp.int32)
    r0 = rout[:, 4].astype(jnp.int32)
    r1 = rout[:, 5].astype(jnp.int32)
    counts = cnt[0, :E].astype(jnp.int32)

    nblocks_e = (counts + BLK - 1) // BLK
    cum_blocks = jnp.cumsum(nblocks_e)
    offs_e = jnp.concatenate(
        [jnp.zeros((1,), jnp.int32), cum_blocks[:-1].astype(jnp.int32)]) * BLK
    pos0 = offs_e[e0] + r0
    pos1 = offs_e[e1] + r1
    tok_iota = jnp.arange(T, dtype=jnp.int32)
    row_token = (jnp.arange(NP, dtype=jnp.int32) % T
                 ).at[pos0].set(tok_iota).at[pos1].set(tok_iota)
    gates_sorted = (jnp.zeros((NP,), jnp.float32)
                    .at[pos0].set(wn1).at[pos1].set(wn2))
    bidx = jnp.arange(NB)
    nlive = cum_blocks[-1].astype(jnp.int32)
    block_expert = jnp.where(
        bidx < nlive,
        jnp.searchsorted(cum_blocks, bidx, side="right"), 0).astype(jnp.int32)
    block_expert = jnp.concatenate([block_expert, nlive[None]])

    x_sorted = _sc_dispatch(x, row_token)
    gates_mat = jnp.broadcast_to(gates_sorted[:, None], (NP, 128))

    eo_sorted = _gmm(block_expert, x_sorted, gates_mat, W1, b1, W2, b2, W3, b3)

    out = _sc_combine(eo_sorted, pos0, pos1)
    return out


# speedup vs baseline: 1.1867x; 1.1867x over previous
import functools

import jax
import jax.numpy as jnp
from jax import lax
from jax.experimental import pallas as pl
from jax.experimental.pallas import tpu as pltpu
from jax.experimental.pallas import tpu_sc as plsc

T = 2048
D = 1024
H = 1024
O = 1024
E = 8
K = 2

BLK = 256
NB = (T * K) // BLK + E
NP = NB * BLK

SC_NC = 2
SC_NS = 16
NW = SC_NC * SC_NS

_SC_MESH = plsc.VectorSubcoreMesh(
    core_axis_name="c", subcore_axis_name="s",
    num_cores=SC_NC, num_subcores=SC_NS)

XROWS = T // NW
CCHUNK = 16
CNCH = XROWS // CCHUNK


def _dispatch_body(x_hbm, pa_hbm, pb_hbm, xs_hbm, ia_v, ib_v, rows_v, s0, s1):
    wid = lax.axis_index("s") * SC_NC + lax.axis_index("c")
    base = wid * XROWS
    pltpu.sync_copy(pa_hbm.at[wid], ia_v)
    pltpu.sync_copy(pb_hbm.at[wid], ib_v)
    pltpu.sync_copy(x_hbm.at[pl.ds(base, XROWS), :], rows_v)
    w0 = pltpu.async_copy(rows_v, xs_hbm.at[ia_v], s0)
    w1 = pltpu.async_copy(rows_v, xs_hbm.at[ib_v], s1)
    w0.wait()
    w1.wait()


def _sc_dispatch(x, p0, p1):
    return pl.kernel(
        _dispatch_body,
        out_type=jax.ShapeDtypeStruct((NP, D), jnp.float32),
        mesh=_SC_MESH,
        scratch_types=[
            pltpu.VMEM((XROWS,), jnp.int32),
            pltpu.VMEM((XROWS,), jnp.int32),
            pltpu.VMEM((XROWS, D), jnp.float32),
            pltpu.SemaphoreType.DMA,
            pltpu.SemaphoreType.DMA,
        ],
    )(x, p0, p1)


def _combine_body(eo_hbm, pa_hbm, pb_hbm, g_hbm, out_hbm, ia_v, ib_v, g_v,
                  a0, a1, b0, b1, o0, o1, gs0, gs1, ws0, ws1):
    wid = lax.axis_index("s") * SC_NC + lax.axis_index("c")
    base = wid * XROWS
    abufs, bbufs, obufs = (a0, a1), (b0, b1), (o0, o1)
    gsems, wsems = (gs0, gs1), (ws0, ws1)
    pltpu.sync_copy(pa_hbm.at[wid], ia_v)
    pltpu.sync_copy(pb_hbm.at[wid], ib_v)
    pltpu.sync_copy(g_hbm.at[pl.ds(base, XROWS), :], g_v)
    ga, gb, wo = [None] * CNCH, [None] * CNCH, [None] * CNCH

    def start_gathers(c):
        ga[c] = pltpu.async_copy(
            eo_hbm.at[ia_v.at[pl.ds(c * CCHUNK, CCHUNK)]],
            abufs[c % 2], gsems[c % 2])
        gb[c] = pltpu.async_copy(
            eo_hbm.at[ib_v.at[pl.ds(c * CCHUNK, CCHUNK)]],
            bbufs[c % 2], gsems[c % 2])

    start_gathers(0)
    for c in range(1, CNCH + 1):
        if c < CNCH:
            if c >= 2:
                wo[c - 2].wait()
            start_gathers(c)
        p = c - 1
        ga[p].wait()
        gb[p].wait()
        av, bv, ov = abufs[p % 2], bbufs[p % 2], obufs[p % 2]
        for r in range(CCHUNK):
            gv = g_v[p * CCHUNK + r, pl.ds(0, 16)]
            g0 = gv[0]
            g1 = gv[1]

            @plsc.parallel_loop(0, O // 16, 1, unroll=8)
            def _fma(j, r=r, g0=g0, g1=g1):
                sl = pl.ds(pl.multiple_of(lax.shift_left(j, 4), 16), 16)
                ov[r, sl] = av[r, sl] * g0 + bv[r, sl] * g1

        wo[p] = pltpu.async_copy(
            ov, out_hbm.at[pl.ds(base + p * CCHUNK, CCHUNK), :],
            wsems[p % 2])
    wo[CNCH - 2].wait()
    wo[CNCH - 1].wait()


def _sc_combine(eo_sorted, p0, p1, rout):
    return pl.kernel(
        _combine_body,
        out_type=jax.ShapeDtypeStruct((T, O), jnp.float32),
        mesh=_SC_MESH,
        scratch_types=[
            pltpu.VMEM((XROWS,), jnp.int32),
            pltpu.VMEM((XROWS,), jnp.int32),
            pltpu.VMEM((XROWS, 128), jnp.float32),
            pltpu.VMEM((CCHUNK, O), jnp.float32),
            pltpu.VMEM((CCHUNK, O), jnp.float32),
            pltpu.VMEM((CCHUNK, O), jnp.float32),
            pltpu.VMEM((CCHUNK, O), jnp.float32),
            pltpu.VMEM((CCHUNK, O), jnp.float32),
            pltpu.VMEM((CCHUNK, O), jnp.float32),
            pltpu.SemaphoreType.DMA,
            pltpu.SemaphoreType.DMA,
            pltpu.SemaphoreType.DMA,
            pltpu.SemaphoreType.DMA,
        ],
    )(eo_sorted, p0, p1, rout)


def _gmm_body(be_ref, x_ref, w1_ref, b1_ref, w2_ref, b2_ref, w3_ref,
              b3_ref, o_ref):
    @pl.when(pl.program_id(0) < be_ref[NB])
    def _():
        x = x_ref[...]
        h1 = jnp.dot(x, w1_ref[0], preferred_element_type=jnp.float32)
        h1 = jnp.maximum(h1 + b1_ref[0], 0.0)
        h2 = jnp.dot(h1, w2_ref[0], preferred_element_type=jnp.float32)
        h2 = jnp.maximum(h2 + b2_ref[0], 0.0)
        eo = jnp.dot(h2, w3_ref[0], preferred_element_type=jnp.float32)
        o_ref[...] = eo + b3_ref[0]


def _gmm(block_expert, x_sorted, W1, b1, W2, b2, W3, b3):
    def rows_map(i, be):
        return (i, 0)

    def w_map(i, be):
        return (be[i], 0, 0)

    def b_map(i, be):
        return (be[i], 0, 0)

    grid_spec = pltpu.PrefetchScalarGridSpec(
        num_scalar_prefetch=1,
        grid=(NB,),
        in_specs=[
            pl.BlockSpec((BLK, D), rows_map),
            pl.BlockSpec((1, D, H), w_map),
            pl.BlockSpec((1, 1, H), b_map),
            pl.BlockSpec((1, H, H), w_map),
            pl.BlockSpec((1, 1, H), b_map),
            pl.BlockSpec((1, H, O), w_map),
            pl.BlockSpec((1, 1, O), b_map),
        ],
        out_specs=pl.BlockSpec((BLK, O), rows_map),
    )
    return pl.pallas_call(
        _gmm_body,
        grid_spec=grid_spec,
        out_shape=jax.ShapeDtypeStruct((NP, O), jnp.float32),
    )(block_expert, x_sorted,
      W1, b1.reshape(E, 1, H), W2, b2.reshape(E, 1, H), W3, b3.reshape(E, 1, O))


BLKR = 512


def _router_body(x_ref, wr_ref, br_ref, ls_ref, out_ref, cnt_ref, carry):
    i = pl.program_id(0)

    @pl.when(i == 0)
    def _():
        carry[...] = jnp.zeros_like(carry)

    logits = jnp.dot(x_ref[...], wr_ref[...],
                     preferred_element_type=jnp.float32) + br_ref[...]
    li = lax.broadcasted_iota(jnp.int32, (BLKR, 128), 1)
    m1 = jnp.max(logits, axis=1)
    a1 = jnp.min(jnp.where(logits == m1[:, None], li, 128), axis=1)
    l2 = jnp.where(li == a1[:, None], -jnp.inf, logits)
    m2 = jnp.max(l2, axis=1)
    a2 = jnp.min(jnp.where(l2 == m2[:, None], li, 128), axis=1)
    z = jnp.sum(jnp.exp(logits - m1[:, None]), axis=1)
    p1 = 1.0 / z
    p2 = jnp.exp(m2 - m1) / z
    den = p1 + p2 + 1e-6
    wn1 = p1 / den
    wn2 = p2 / den
    oh0 = (li == a1[:, None]).astype(jnp.float32)
    oh1 = (li == a2[:, None]).astype(jnp.float32)
    s = jnp.dot(ls_ref[...], oh0 + oh1,
                preferred_element_type=jnp.float32) + carry[0:1, :]
    r0 = jnp.sum(s * oh0, axis=1)
    r1 = jnp.sum((s + oh0) * oh1, axis=1)
    colsum = jnp.sum(oh0 + oh1, axis=0, keepdims=True)
    carry[...] = carry[...] + colsum
    out = jnp.where(li == 0, wn1[:, None],
          jnp.where(li == 1, wn2[:, None],
          jnp.where(li == 2, a1[:, None].astype(jnp.float32),
          jnp.where(li == 3, a2[:, None].astype(jnp.float32),
          jnp.where(li == 4, r0[:, None],
          jnp.where(li == 5, r1[:, None], 0.0))))))
    out_ref[...] = out

    @pl.when(i == pl.num_programs(0) - 1)
    def _():
        cnt_ref[...] = carry[...]


def _router(x, Wr, br):
    wr_pad = jnp.zeros((D, 128), jnp.float32).at[:, :E].set(Wr)
    br_pad = jnp.full((1, 128), -1e30, jnp.float32).at[0, :E].set(br)
    ls = jnp.tril(jnp.ones((BLKR, BLKR), jnp.float32), -1)
    grid_spec = pltpu.PrefetchScalarGridSpec(
        num_scalar_prefetch=0,
        grid=(T // BLKR,),
        in_specs=[
            pl.BlockSpec((BLKR, D), lambda i: (i, 0)),
            pl.BlockSpec((D, 128), lambda i: (0, 0)),
            pl.BlockSpec((1, 128), lambda i: (0, 0)),
            pl.BlockSpec((BLKR, BLKR), lambda i: (0, 0)),
        ],
        out_specs=[
            pl.BlockSpec((BLKR, 128), lambda i: (i, 0)),
            pl.BlockSpec((1, 128), lambda i: (0, 0)),
        ],
        scratch_shapes=[pltpu.VMEM((1, 128), jnp.float32)],
    )
    return pl.pallas_call(
        _router_body,
        grid_spec=grid_spec,
        out_shape=[
            jax.ShapeDtypeStruct((T, 128), jnp.float32),
            jax.ShapeDtypeStruct((1, 128), jnp.float32),
        ],
    )(x, wr_pad, br_pad, ls)


def kernel(x, Wr, br, W1, b1, W2, b2, W3, b3):
    rout, cnt = _router(x, Wr, br)
    e0 = rout[:, 2].astype(jnp.int32)
    e1 = rout[:, 3].astype(jnp.int32)
    r0 = rout[:, 4].astype(jnp.int32)
    r1 = rout[:, 5].astype(jnp.int32)
    counts = cnt[0, :E].astype(jnp.int32)

    nblocks_e = (counts + BLK - 1) // BLK
    cum_blocks = jnp.cumsum(nblocks_e)
    offs_e = jnp.concatenate(
        [jnp.zeros((1,), jnp.int32), cum_blocks[:-1].astype(jnp.int32)]) * BLK
    pos0 = (offs_e[e0] + r0).reshape(NW, XROWS)
    pos1 = (offs_e[e1] + r1).reshape(NW, XROWS)
    bidx = jnp.arange(NB)
    nlive = cum_blocks[-1].astype(jnp.int32)
    block_expert = jnp.where(
        bidx < nlive,
        jnp.searchsorted(cum_blocks, bidx, side="right"), 0).astype(jnp.int32)
    block_expert = jnp.concatenate([block_expert, nlive[None]])

    x_sorted = _sc_dispatch(x, pos0, pos1)

    eo_sorted = _gmm(block_expert, x_sorted, W1, b1, W2, b2, W3, b3)

    out = _sc_combine(eo_sorted, pos0, pos1, rout)
    return out
